# baseline (speedup 1.0000x reference)
import functools

import jax
import jax.numpy as jnp
from jax import lax
from jax.experimental import pallas as pl
from jax.experimental.pallas import tpu as pltpu
from jax.experimental.pallas import tpu_sc as plsc

N = 10000
E = 320000
D = 128
L = 2
EPS = 1e-5

NP = 10240
DH = D // 2
CHUNK = 128
T_DEG = 84
T_AGG = 168
BATCH = 4
NB = T_AGG // BATCH
T_IDX = (NB + 2) * BATCH
EPAD = 16 * T_AGG * CHUNK
ROWS_PER_TILE = NP // 16
ZCOPIES = ROWS_PER_TILE // CHUNK
NBUF = 4

_mesh = plsc.VectorSubcoreMesh(core_axis_name="c", subcore_axis_name="s")
_sc_params = pltpu.CompilerParams(use_tc_tiling_on_sc=False)


@functools.partial(
    pl.kernel,
    out_type=jax.ShapeDtypeStruct((2, NP, 16), jnp.float32),
    mesh=_mesh,
    compiler_params=_sc_params,
    scratch_types=[
        pltpu.VMEM((T_DEG, CHUNK), jnp.int32),
        pltpu.VMEM((CHUNK, 16), jnp.float32),
        pltpu.VMEM((CHUNK, 16), jnp.float32),
        pltpu.VMEM_SHARED((NP, 16), jnp.float32),
        pltpu.SemaphoreType.DMA,
    ],
)
def _sc_degree(dst_hbm, ones_hbm, zeros_hbm, out_hbm,
               idx_d, ones_v, zero_v, acc_sh, sem):
  cid = lax.axis_index("c")
  sid = lax.axis_index("s")
  wid = cid * 16 + sid
  pltpu.sync_copy(dst_hbm.at[wid], idx_d)
  pltpu.sync_copy(ones_hbm, ones_v)
  pltpu.sync_copy(zeros_hbm, zero_v)
  base = sid * ROWS_PER_TILE
  for k in range(ZCOPIES):
    pltpu.sync_copy(zero_v, acc_sh.at[pl.ds(base + k * CHUNK, CHUNK)])
  plsc.subcore_barrier()

  @pl.loop(0, T_DEG, step=NBUF)
  def _(t0):
    cps = [
        pltpu.async_copy(ones_v, acc_sh.at[idx_d.at[t0 + i]], sem, add=True)
        for i in range(NBUF)
    ]
    for cp in cps:
      cp.wait()

  plsc.subcore_barrier()
  for k in range(ZCOPIES):
    r = base + k * CHUNK
    pltpu.sync_copy(acc_sh.at[pl.ds(r, CHUNK)], out_hbm.at[cid, pl.ds(r, CHUNK)])


@functools.partial(
    pl.kernel,
    out_type=jax.ShapeDtypeStruct((2, NP, DH), jnp.float32),
    mesh=_mesh,
    compiler_params=_sc_params,
    scratch_types=[
        pltpu.VMEM((3, BATCH, CHUNK), jnp.int32),
        pltpu.VMEM((3, BATCH, CHUNK), jnp.int32),
        pltpu.VMEM((2 * BATCH, CHUNK, DH), jnp.float32),
        pltpu.VMEM((CHUNK, DH), jnp.float32),
        pltpu.VMEM_SHARED((NP, DH), jnp.float32),
        pltpu.SemaphoreType.DMA,
        pltpu.SemaphoreType.DMA,
        pltpu.SemaphoreType.DMA,
    ],
)
def _sc_aggregate(g2_hbm, srcx_hbm, dstx_hbm, zeros_hbm, out_hbm,
                  idxs, idxd, rows_v, zero_v, acc_sh, isem, gsem, ssem):
  cid = lax.axis_index("c")
  sid = lax.axis_index("s")
  src_t = srcx_hbm.at[cid, sid]
  dst_t = dstx_hbm.at[sid]
  pltpu.sync_copy(zeros_hbm, zero_v)
  base = sid * ROWS_PER_TILE
  for k in range(ZCOPIES):
    pltpu.sync_copy(zero_v, acc_sh.at[pl.ds(base + k * CHUNK, CHUNK)])

  def idx_issue(b, r):
    pltpu.async_copy(src_t.at[pl.ds(b * BATCH, BATCH)], idxs.at[r], isem)
    pltpu.async_copy(dst_t.at[pl.ds(b * BATCH, BATCH)], idxd.at[r], isem)

  def idx_drain(b, r):
    pltpu.make_async_copy(src_t.at[pl.ds(b * BATCH, BATCH)], idxs.at[r],
                          isem).wait()
    pltpu.make_async_copy(dst_t.at[pl.ds(b * BATCH, BATCH)], idxd.at[r],
                          isem).wait()

  idx_issue(0, 0)
  idx_issue(1, 1)
  idx_drain(0, 0)
  plsc.subcore_barrier()
  for i in range(BATCH):
    pltpu.async_copy(g2_hbm.at[idxs.at[0, i]], rows_v.at[i], gsem)
  for i in range(BATCH):
    pltpu.async_copy(zero_v, acc_sh.at[idxd.at[0, i]], ssem, add=True)

  def _slot(b, g, r):
    t0 = b * BATCH
    rn, rn2 = (r + 1) % 3, (r + 2) % 3
    for i in range(BATCH):
      pltpu.make_async_copy(g2_hbm.at[idxs.at[r, i]],
                            rows_v.at[g * BATCH + i], gsem).wait()
    for i in range(BATCH):
      pltpu.make_async_copy(rows_v.at[(1 - g) * BATCH + i],
                            acc_sh.at[idxd.at[r, i]], ssem).wait()
    for i in range(BATCH):
      pltpu.async_copy(rows_v.at[g * BATCH + i], acc_sh.at[idxd.at[r, i]],
                       ssem, add=True)
    idx_drain(b + 1, rn)
    for i in range(BATCH):
      pltpu.async_copy(g2_hbm.at[idxs.at[rn, i]],
                       rows_v.at[(1 - g) * BATCH + i], gsem)
    idx_issue(b + 2, rn2)

  @pl.loop(0, NB, step=6)
  def _(b0):
    for j in range(6):
      _slot(b0 + j, j % 2, j % 3)

  for i in range(BATCH):
    pltpu.make_async_copy(rows_v.at[BATCH + i], acc_sh.at[idxd.at[0, i]],
                          ssem).wait()
  for i in range(BATCH):
    pltpu.make_async_copy(g2_hbm.at[idxs.at[0, i]], rows_v.at[i], gsem).wait()
  idx_drain(NB + 1, (NB + 1) % 3)

  plsc.subcore_barrier()
  for k in range(ZCOPIES):
    r = base + k * CHUNK
    pltpu.sync_copy(acc_sh.at[pl.ds(r, CHUNK)], out_hbm.at[cid, pl.ds(r, CHUNK)])


TC_BLK = 2048


def _dinv_block(degp):
  deg = degp[0, :, 0:1] + degp[1, :, 0:1]
  return jnp.where(deg > 0, lax.rsqrt(jnp.maximum(deg, 1e-12)), 0.0)


def _tc_first_body(x_ref, w_ref, degp_ref, g_ref):
  dinv = _dinv_block(degp_ref[...])
  g_ref[...] = dinv * jnp.dot(x_ref[...], w_ref[...],
                              preferred_element_type=jnp.float32)


def _tc_mid_body(agg_ref, degp_ref, bias_ref, w_ref, g_ref):
  dinv = _dinv_block(degp_ref[...])
  agg = jnp.concatenate([agg_ref[0], agg_ref[1]], axis=1)
  h = jnp.maximum(dinv * agg + bias_ref[...], 0.0)
  g_ref[...] = dinv * jnp.dot(h, w_ref[...], preferred_element_type=jnp.float32)


def _tc_last_body(agg_ref, degp_ref, bias_ref, out_ref):
  dinv = _dinv_block(degp_ref[...])
  agg = jnp.concatenate([agg_ref[0], agg_ref[1]], axis=1)
  out_ref[...] = jnp.maximum(dinv * agg + bias_ref[...], 0.0)


_row_spec = pl.BlockSpec((TC_BLK, D), lambda i: (i, 0))
_split_spec = pl.BlockSpec((2, TC_BLK, DH), lambda i: (0, i, 0))
_degp_spec = pl.BlockSpec((2, TC_BLK, 16), lambda i: (0, i, 0))
_w_spec = pl.BlockSpec((D, D), lambda i: (0, 0))
_bias_spec = pl.BlockSpec((1, D), lambda i: (0, 0))
_out_struct = jax.ShapeDtypeStruct((NP, D), jnp.float32)

_tc_first = pl.pallas_call(
    _tc_first_body, grid=(NP // TC_BLK,),
    in_specs=[_row_spec, _w_spec, _degp_spec],
    out_specs=_row_spec, out_shape=_out_struct)

_tc_mid = pl.pallas_call(
    _tc_mid_body, grid=(NP // TC_BLK,),
    in_specs=[_split_spec, _degp_spec, _bias_spec, _w_spec],
    out_specs=_row_spec, out_shape=_out_struct)

_tc_last = pl.pallas_call(
    _tc_last_body, grid=(NP // TC_BLK,),
    in_specs=[_split_spec, _degp_spec, _bias_spec],
    out_specs=_row_spec, out_shape=_out_struct)


@jax.jit
def _run(x, edge_index, W, b, gamma, beta, running_mean, running_var):
  s = gamma * lax.rsqrt(running_var + EPS)
  t = beta - running_mean * s
  W_eff = W * s[:, None, :]
  bias_eff = (b * s + t)[:, None, :]

  pad = EPAD - E
  src = jnp.concatenate([edge_index[0], jnp.zeros((pad,), jnp.int32)])
  dst = jnp.concatenate([edge_index[1], jnp.full((pad,), N, jnp.int32)])
  ext = T_IDX - T_AGG
  src2 = jnp.concatenate(
      [(src * 2).reshape(16, T_AGG, CHUNK),
       jnp.zeros((16, ext, CHUNK), jnp.int32)], axis=1)
  srcx = jnp.stack([src2, src2 + 1])
  dstx = jnp.concatenate(
      [dst.reshape(16, T_AGG, CHUNK),
       jnp.full((16, ext, CHUNK), N, jnp.int32)], axis=1)
  dst_deg = dst.reshape(32, T_DEG, CHUNK)

  xp = jnp.zeros((NP, D), jnp.float32).at[:N].set(x)
  ones16 = jnp.ones((CHUNK, 16), jnp.float32)
  zeros16 = jnp.zeros((CHUNK, 16), jnp.float32)
  zerosdh = jnp.zeros((CHUNK, DH), jnp.float32)

  degp = _sc_degree(dst_deg, ones16, zeros16)
  g = _tc_first(xp, W_eff[0], degp)
  agg = _sc_aggregate(g.reshape(2 * NP, DH), srcx, dstx, zerosdh)
  g = _tc_mid(agg, degp, bias_eff[0], W_eff[1])
  agg = _sc_aggregate(g.reshape(2 * NP, DH), srcx, dstx, zerosdh)
  out = _tc_last(agg, degp, bias_eff[1])
  return out[:N]


def kernel(x, edge_index, W, b, gamma, beta, running_mean, running_var):
  return _run(x, edge_index, W, b, gamma, beta, running_mean, running_var)


# speedup vs baseline: 2.8955x; 2.8955x over previous
import functools

import jax
import jax.numpy as jnp
from jax import lax
from jax.experimental import pallas as pl
from jax.experimental.pallas import tpu as pltpu
from jax.experimental.pallas import tpu_sc as plsc

N = 10000
E = 320000
D = 128
L = 2
EPS = 1e-5

NP = 10240
DH = D // 2
CHUNK = 128
T_DEG = 80
T_AGG = 160
EPAD = 32 * T_DEG * CHUNK
ROWS_PER_TILE = NP // 16
ZCOPIES = ROWS_PER_TILE // CHUNK
NBUF = 4

_mesh = plsc.VectorSubcoreMesh(core_axis_name="c", subcore_axis_name="s")
_sc_params = pltpu.CompilerParams(use_tc_tiling_on_sc=False)


@functools.partial(
    pl.kernel,
    out_type=jax.ShapeDtypeStruct((2, NP, 16), jnp.float32),
    mesh=_mesh,
    compiler_params=_sc_params,
    scratch_types=[
        pltpu.VMEM((T_DEG, CHUNK), jnp.int32),
        pltpu.VMEM((CHUNK, 16), jnp.float32),
        pltpu.VMEM((CHUNK, 16), jnp.float32),
        pltpu.VMEM_SHARED((NP, 16), jnp.float32),
        pltpu.SemaphoreType.DMA,
    ],
)
def _sc_degree(dst_hbm, ones_hbm, zeros_hbm, out_hbm,
               idx_d, ones_v, zero_v, acc_sh, sem):
  cid = lax.axis_index("c")
  sid = lax.axis_index("s")
  wid = cid * 16 + sid
  pltpu.sync_copy(dst_hbm.at[wid], idx_d)
  pltpu.sync_copy(ones_hbm, ones_v)
  pltpu.sync_copy(zeros_hbm, zero_v)
  base = sid * ROWS_PER_TILE
  for k in range(ZCOPIES):
    pltpu.sync_copy(zero_v, acc_sh.at[pl.ds(base + k * CHUNK, CHUNK)])
  plsc.subcore_barrier()

  @pl.loop(0, T_DEG, step=NBUF)
  def _(t0):
    cps = [
        pltpu.async_copy(ones_v, acc_sh.at[idx_d.at[t0 + i]], sem, add=True)
        for i in range(NBUF)
    ]
    for cp in cps:
      cp.wait()

  plsc.subcore_barrier()
  for k in range(ZCOPIES):
    r = base + k * CHUNK
    pltpu.sync_copy(acc_sh.at[pl.ds(r, CHUNK)], out_hbm.at[cid, pl.ds(r, CHUNK)])


@functools.partial(
    pl.kernel,
    out_type=jax.ShapeDtypeStruct((2, NP, DH), jnp.float32),
    mesh=_mesh,
    compiler_params=_sc_params,
    scratch_types=[
        pltpu.VMEM((T_AGG, CHUNK), jnp.int32),
        pltpu.VMEM((T_AGG, CHUNK), jnp.int32),
        pltpu.VMEM((NBUF, CHUNK, DH), jnp.float32),
        pltpu.VMEM((CHUNK, DH), jnp.float32),
        pltpu.VMEM_SHARED((NP, DH), jnp.float32),
        pltpu.SemaphoreType.DMA,
        pltpu.SemaphoreType.DMA,
    ],
)
def _sc_aggregate(g2_hbm, srcx_hbm, dstx_hbm, zeros_hbm, out_hbm,
                  idx_s, idx_d, rows_v, zero_v, acc_sh, gsem, ssem):
  cid = lax.axis_index("c")
  sid = lax.axis_index("s")
  pltpu.sync_copy(srcx_hbm.at[cid, sid], idx_s)
  pltpu.sync_copy(dstx_hbm.at[sid], idx_d)
  pltpu.sync_copy(zeros_hbm, zero_v)
  base = sid * ROWS_PER_TILE
  for k in range(ZCOPIES):
    pltpu.sync_copy(zero_v, acc_sh.at[pl.ds(base + k * CHUNK, CHUNK)])
  plsc.subcore_barrier()

  @pl.loop(0, T_AGG, step=NBUF)
  def _(t0):
    for i in range(NBUF):
      pltpu.async_copy(g2_hbm.at[idx_s.at[t0 + i]], rows_v.at[i], gsem)
    for i in range(NBUF):
      pltpu.make_async_copy(g2_hbm.at[idx_s.at[t0 + i]], rows_v.at[i],
                            gsem).wait()
    for i in range(NBUF):
      pltpu.async_copy(rows_v.at[i], acc_sh.at[idx_d.at[t0 + i]], ssem,
                       add=True)
    for i in range(NBUF):
      pltpu.make_async_copy(rows_v.at[i], acc_sh.at[idx_d.at[t0 + i]],
                            ssem).wait()

  plsc.subcore_barrier()
  for k in range(ZCOPIES):
    r = base + k * CHUNK
    pltpu.sync_copy(acc_sh.at[pl.ds(r, CHUNK)], out_hbm.at[cid, pl.ds(r, CHUNK)])


TC_BLK = 2048


def _dinv_block(degp):
  deg = degp[0, :, 0:1] + degp[1, :, 0:1]
  return jnp.where(deg > 0, lax.rsqrt(jnp.maximum(deg, 1e-12)), 0.0)


def _tc_first_body(x_ref, w_ref, degp_ref, g_ref):
  dinv = _dinv_block(degp_ref[...])
  g_ref[...] = dinv * jnp.dot(x_ref[...], w_ref[...],
                              preferred_element_type=jnp.float32)


def _tc_mid_body(agg_ref, degp_ref, bias_ref, w_ref, g_ref):
  dinv = _dinv_block(degp_ref[...])
  agg = jnp.concatenate([agg_ref[0], agg_ref[1]], axis=1)
  h = jnp.maximum(dinv * agg + bias_ref[...], 0.0)
  g_ref[...] = dinv * jnp.dot(h, w_ref[...], preferred_element_type=jnp.float32)


def _tc_last_body(agg_ref, degp_ref, bias_ref, out_ref):
  dinv = _dinv_block(degp_ref[...])
  agg = jnp.concatenate([agg_ref[0], agg_ref[1]], axis=1)
  out_ref[...] = jnp.maximum(dinv * agg + bias_ref[...], 0.0)


_row_spec = pl.BlockSpec((TC_BLK, D), lambda i: (i, 0))
_split_spec = pl.BlockSpec((2, TC_BLK, DH), lambda i: (0, i, 0))
_degp_spec = pl.BlockSpec((2, TC_BLK, 16), lambda i: (0, i, 0))
_w_spec = pl.BlockSpec((D, D), lambda i: (0, 0))
_bias_spec = pl.BlockSpec((1, D), lambda i: (0, 0))
_out_struct = jax.ShapeDtypeStruct((NP, D), jnp.float32)

_tc_first = pl.pallas_call(
    _tc_first_body, grid=(NP // TC_BLK,),
    in_specs=[_row_spec, _w_spec, _degp_spec],
    out_specs=_row_spec, out_shape=_out_struct)

_tc_mid = pl.pallas_call(
    _tc_mid_body, grid=(NP // TC_BLK,),
    in_specs=[_split_spec, _degp_spec, _bias_spec, _w_spec],
    out_specs=_row_spec, out_shape=_out_struct)

_tc_last = pl.pallas_call(
    _tc_last_body, grid=(NP // TC_BLK,),
    in_specs=[_split_spec, _degp_spec, _bias_spec],
    out_specs=_row_spec, out_shape=_out_struct)


@jax.jit
def _run(x, edge_index, W, b, gamma, beta, running_mean, running_var):
  s = gamma * lax.rsqrt(running_var + EPS)
  t = beta - running_mean * s
  W_eff = W * s[:, None, :]
  bias_eff = (b * s + t)[:, None, :]

  pad = EPAD - E
  src = jnp.concatenate([edge_index[0], jnp.zeros((pad,), jnp.int32)])
  dst = jnp.concatenate([edge_index[1], jnp.full((pad,), N, jnp.int32)])
  src2 = src * 2
  srcx = jnp.stack([src2, src2 + 1]).reshape(2, 16, T_AGG, CHUNK)
  dstx = dst.reshape(16, T_AGG, CHUNK)
  dst_deg = dst.reshape(32, T_DEG, CHUNK)

  xp = jnp.zeros((NP, D), jnp.float32).at[:N].set(x)
  ones16 = jnp.ones((CHUNK, 16), jnp.float32)
  zeros16 = jnp.zeros((CHUNK, 16), jnp.float32)
  zerosdh = jnp.zeros((CHUNK, DH), jnp.float32)

  degp = _sc_degree(dst_deg, ones16, zeros16)
  g = _tc_first(xp, W_eff[0], degp)
  agg = _sc_aggregate(g.reshape(2 * NP, DH), srcx, dstx, zerosdh)
  g = _tc_mid(agg, degp, bias_eff[0], W_eff[1])
  agg = _sc_aggregate(g.reshape(2 * NP, DH), srcx, dstx, zerosdh)
  out = _tc_last(agg, degp, bias_eff[1])
  return out[:N]


def kernel(x, edge_index, W, b, gamma, beta, running_mean, running_var):
  return _run(x, edge_index, W, b, gamma, beta, running_mean, running_var)


# speedup vs baseline: 3.0192x; 1.0427x over previous
import functools

import jax
import jax.numpy as jnp
from jax import lax
from jax.experimental import pallas as pl
from jax.experimental.pallas import tpu as pltpu
from jax.experimental.pallas import tpu_sc as plsc

N = 10000
E = 320000
D = 128
L = 2
EPS = 1e-5

NP = 10240
DH = D // 2
CHUNK = 128
T_DEG = 80
T_AGG = 160
EPAD = 32 * T_DEG * CHUNK
ROWS_PER_TILE = NP // 16
ZCOPIES = ROWS_PER_TILE // CHUNK
NBUF = 4

_mesh = plsc.VectorSubcoreMesh(core_axis_name="c", subcore_axis_name="s")
_sc_params = pltpu.CompilerParams(use_tc_tiling_on_sc=False)


@functools.partial(
    pl.kernel,
    out_type=jax.ShapeDtypeStruct((2, NP, 16), jnp.float32),
    mesh=_mesh,
    compiler_params=_sc_params,
    scratch_types=[
        pltpu.VMEM((T_DEG, CHUNK), jnp.int32),
        pltpu.VMEM((CHUNK, 16), jnp.float32),
        pltpu.VMEM((CHUNK, 16), jnp.float32),
        pltpu.VMEM_SHARED((NP, 16), jnp.float32),
        pltpu.SemaphoreType.DMA,
    ],
)
def _sc_degree(dst_hbm, ones_hbm, zeros_hbm, out_hbm,
               idx_d, ones_v, zero_v, acc_sh, sem):
  cid = lax.axis_index("c")
  sid = lax.axis_index("s")
  wid = cid * 16 + sid
  pltpu.sync_copy(dst_hbm.at[wid], idx_d)
  pltpu.sync_copy(ones_hbm, ones_v)
  pltpu.sync_copy(zeros_hbm, zero_v)
  base = sid * ROWS_PER_TILE
  for k in range(ZCOPIES):
    pltpu.sync_copy(zero_v, acc_sh.at[pl.ds(base + k * CHUNK, CHUNK)])
  plsc.subcore_barrier()

  @pl.loop(0, T_DEG, step=NBUF)
  def _(t0):
    cps = [
        pltpu.async_copy(ones_v, acc_sh.at[idx_d.at[t0 + i]], sem, add=True)
        for i in range(NBUF)
    ]
    for cp in cps:
      cp.wait()

  plsc.subcore_barrier()
  for k in range(ZCOPIES):
    r = base + k * CHUNK
    pltpu.sync_copy(acc_sh.at[pl.ds(r, CHUNK)], out_hbm.at[cid, pl.ds(r, CHUNK)])


@functools.partial(
    pl.kernel,
    out_type=jax.ShapeDtypeStruct((2, NP, DH), jnp.float32),
    mesh=_mesh,
    compiler_params=_sc_params,
    scratch_types=[
        pltpu.VMEM((T_AGG, CHUNK), jnp.int32),
        pltpu.VMEM((T_AGG, CHUNK), jnp.int32),
        pltpu.VMEM((NBUF, CHUNK, DH), jnp.float32),
        pltpu.VMEM((CHUNK, DH), jnp.float32),
        pltpu.VMEM_SHARED((NP, DH), jnp.float32),
        pltpu.SemaphoreType.DMA,
        pltpu.SemaphoreType.DMA,
    ],
)
def _sc_aggregate(g2_hbm, srcx_hbm, dstx_hbm, zeros_hbm, out_hbm,
                  idx_s, idx_d, rows_v, zero_v, acc_sh, gsem, ssem):
  cid = lax.axis_index("c")
  sid = lax.axis_index("s")
  pltpu.sync_copy(srcx_hbm.at[cid, sid], idx_s)
  pltpu.sync_copy(dstx_hbm.at[sid], idx_d)
  pltpu.sync_copy(zeros_hbm, zero_v)
  base = sid * ROWS_PER_TILE
  for k in range(ZCOPIES):
    pltpu.sync_copy(zero_v, acc_sh.at[pl.ds(base + k * CHUNK, CHUNK)])
  plsc.subcore_barrier()

  @pl.loop(0, T_AGG, step=NBUF)
  def _(t0):
    for i in range(NBUF):
      pltpu.async_copy(g2_hbm.at[idx_s.at[t0 + i]], rows_v.at[i], gsem)
    scps = []
    for i in range(NBUF):
      pltpu.make_async_copy(g2_hbm.at[idx_s.at[t0 + i]], rows_v.at[i],
                            gsem).wait()
      scps.append(
          pltpu.async_copy(rows_v.at[i], acc_sh.at[idx_d.at[t0 + i]], ssem,
                           add=True))
    for cp in scps:
      cp.wait()

  plsc.subcore_barrier()
  for k in range(ZCOPIES):
    r = base + k * CHUNK
    pltpu.sync_copy(acc_sh.at[pl.ds(r, CHUNK)], out_hbm.at[cid, pl.ds(r, CHUNK)])


TC_BLK = 2048


def _dinv_block(degp):
  deg = degp[0, :, 0:1] + degp[1, :, 0:1]
  return jnp.where(deg > 0, lax.rsqrt(jnp.maximum(deg, 1e-12)), 0.0)


def _tc_first_body(x_ref, w_ref, degp_ref, g_ref):
  dinv = _dinv_block(degp_ref[...])
  g_ref[...] = dinv * jnp.dot(x_ref[...], w_ref[...],
                              preferred_element_type=jnp.float32)


def _tc_mid_body(agg_ref, degp_ref, bias_ref, w_ref, g_ref):
  dinv = _dinv_block(degp_ref[...])
  agg = jnp.concatenate([agg_ref[0], agg_ref[1]], axis=1)
  h = jnp.maximum(dinv * agg + bias_ref[...], 0.0)
  g_ref[...] = dinv * jnp.dot(h, w_ref[...], preferred_element_type=jnp.float32)


def _tc_last_body(agg_ref, degp_ref, bias_ref, out_ref):
  dinv = _dinv_block(degp_ref[...])
  agg = jnp.concatenate([agg_ref[0], agg_ref[1]], axis=1)
  out_ref[...] = jnp.maximum(dinv * agg + bias_ref[...], 0.0)


_row_spec = pl.BlockSpec((TC_BLK, D), lambda i: (i, 0))
_split_spec = pl.BlockSpec((2, TC_BLK, DH), lambda i: (0, i, 0))
_degp_spec = pl.BlockSpec((2, TC_BLK, 16), lambda i: (0, i, 0))
_w_spec = pl.BlockSpec((D, D), lambda i: (0, 0))
_bias_spec = pl.BlockSpec((1, D), lambda i: (0, 0))
_out_struct = jax.ShapeDtypeStruct((NP, D), jnp.float32)

_tc_first = pl.pallas_call(
    _tc_first_body, grid=(NP // TC_BLK,),
    in_specs=[_row_spec, _w_spec, _degp_spec],
    out_specs=_row_spec, out_shape=_out_struct)

_tc_mid = pl.pallas_call(
    _tc_mid_body, grid=(NP // TC_BLK,),
    in_specs=[_split_spec, _degp_spec, _bias_spec, _w_spec],
    out_specs=_row_spec, out_shape=_out_struct)

_tc_last = pl.pallas_call(
    _tc_last_body, grid=(NP // TC_BLK,),
    in_specs=[_split_spec, _degp_spec, _bias_spec],
    out_specs=_row_spec, out_shape=_out_struct)


@jax.jit
def _run(x, edge_index, W, b, gamma, beta, running_mean, running_var):
  s = gamma * lax.rsqrt(running_var + EPS)
  t = beta - running_mean * s
  W_eff = W * s[:, None, :]
  bias_eff = (b * s + t)[:, None, :]

  pad = EPAD - E
  src = jnp.concatenate([edge_index[0], jnp.zeros((pad,), jnp.int32)])
  dst = jnp.concatenate([edge_index[1], jnp.full((pad,), N, jnp.int32)])
  src2 = src * 2
  srcx = jnp.stack([src2, src2 + 1]).reshape(2, 16, T_AGG, CHUNK)
  dstx = dst.reshape(16, T_AGG, CHUNK)
  dst_deg = dst.reshape(32, T_DEG, CHUNK)

  xp = jnp.zeros((NP, D), jnp.float32).at[:N].set(x)
  ones16 = jnp.ones((CHUNK, 16), jnp.float32)
  zeros16 = jnp.zeros((CHUNK, 16), jnp.float32)
  zerosdh = jnp.zeros((CHUNK, DH), jnp.float32)

  degp = _sc_degree(dst_deg, ones16, zeros16)
  g = _tc_first(xp, W_eff[0], degp)
  agg = _sc_aggregate(g.reshape(2 * NP, DH), srcx, dstx, zerosdh)
  g = _tc_mid(agg, degp, bias_eff[0], W_eff[1])
  agg = _sc_aggregate(g.reshape(2 * NP, DH), srcx, dstx, zerosdh)
  out = _tc_last(agg, degp, bias_eff[1])
  return out[:N]


def kernel(x, edge_index, W, b, gamma, beta, running_mean, running_var):
  return _run(x, edge_index, W, b, gamma, beta, running_mean, running_var)
